# Initial kernel scaffold; baseline (speedup 1.0000x reference)
import jax
import jax.numpy as jnp
from jax.experimental import pallas as pl


def kernel(x, pre_h, edge_index, edge_attr, W_enc, b_enc, W_M, b_M, W_U, b_U, W_dec, b_dec, W_ter, b_ter):
    raise NotImplementedError("write your pallas kernel here")


# speedup vs baseline: 1.5207x; 1.5207x over previous
import functools

import jax
import jax.numpy as jnp
from jax import lax
from jax.experimental import pallas as pl
from jax.experimental.pallas import tpu as pltpu
from jax.experimental.pallas import tpu_sc as plsc

N = 10000
E = 320000
H = 128

NC = 2
NS = 16
L = 16

NHC = 8
NG = 2
NH2 = N // 2
EG = E // NG
CK = 640
NCHUNK = EG // CK
TROWS = NH2 + 8


def _enc_body(x_ref, ph_ref, we_ref, be_ref, wm_ref, bm_ref, z_ref, a2_ref, b_ref):
    xw = x_ref[...] * we_ref[0:1, :]
    z = jnp.dot(ph_ref[...], we_ref[1:, :], preferred_element_type=jnp.float32)
    z = jnp.maximum(z + xw + be_ref[...], 0.0)
    z_ref[...] = z
    a2_ref[...] = jnp.dot(z, wm_ref[:H, :], preferred_element_type=jnp.float32) + bm_ref[...]
    b_ref[...] = jnp.dot(z, wm_ref[H:2 * H, :], preferred_element_type=jnp.float32)


def _encode(x, pre_h, W_enc, b_enc, W_M, b_M):
    return pl.pallas_call(
        _enc_body,
        out_shape=[
            jax.ShapeDtypeStruct((N, H), jnp.float32),
            jax.ShapeDtypeStruct((N, H), jnp.float32),
            jax.ShapeDtypeStruct((N, H), jnp.float32),
        ],
    )(x, pre_h, W_enc, b_enc.reshape(1, H), W_M, b_M.reshape(1, H))


def _sc_body(b8_hbm, src_hbm, dst_hbm, attr_hbm, w_hbm, out_hbm,
             tab, idx_v, rows_v, dst_v, attr_v, w_v, sem):
    c = lax.axis_index("c")
    s = lax.axis_index("s")
    hc = s % NHC
    g = s // NHC
    nh = c
    nbase = nh * NH2

    pltpu.sync_copy(w_hbm, w_v)
    wvec = w_v[hc, :]

    def init(i, _):
        tab[i, :] = jnp.full((L,), -jnp.inf, jnp.float32)
        return _
    lax.fori_loop(0, TROWS, init, None)

    def chunk(ci, _):
        ebase = g * EG + ci * CK
        pltpu.sync_copy(src_hbm.at[pl.ds(ebase, CK)], idx_v)
        pltpu.sync_copy(dst_hbm.at[pl.ds(ebase, CK)], dst_v)
        pltpu.sync_copy(attr_hbm.at[pl.ds(ebase, CK)], attr_v)

        def mkidx(j, _):
            idx_v[pl.ds(j * L, L)] = idx_v[pl.ds(j * L, L)] * NHC + hc
            return _
        lax.fori_loop(0, CK // L, mkidx, None)

        pltpu.async_copy(b8_hbm.at[idx_v], rows_v, sem).wait()

        def grp(j, _):
            dst16 = dst_v[pl.ds(j * L, L)]
            att16 = attr_v[pl.ds(j * L, L)]
            for l in range(L):
                d = dst16[l]
                a = att16[l]
                li = d - nbase
                ok = jnp.logical_and(li >= 0, li < NH2)
                li = jnp.where(ok, li, NH2)
                v = rows_v[j * L + l, :] + a * wvec
                tab[li, :] = jnp.maximum(tab[li, :], v)
            return _
        lax.fori_loop(0, CK // L, grp, None)
        return _
    lax.fori_loop(0, NCHUNK, chunk, None)

    pltpu.sync_copy(tab.at[pl.ds(0, NH2)],
                    out_hbm.at[g, pl.ds(nbase, NH2), pl.ds(hc * L, L)])


def _segmax(b8, src, dst, attr, w):
    mesh = plsc.VectorSubcoreMesh(core_axis_name="c", subcore_axis_name="s",
                                  num_cores=NC, num_subcores=NS)
    f = pl.kernel(
        _sc_body,
        out_type=jax.ShapeDtypeStruct((NG, N, H), jnp.float32),
        mesh=mesh,
        compiler_params=pltpu.CompilerParams(use_tc_tiling_on_sc=False),
        scratch_types=[
            pltpu.VMEM((TROWS, L), jnp.float32),
            pltpu.VMEM((CK,), jnp.int32),
            pltpu.VMEM((CK, L), jnp.float32),
            pltpu.VMEM((CK,), jnp.int32),
            pltpu.VMEM((CK,), jnp.float32),
            pltpu.VMEM((NHC, L), jnp.float32),
            pltpu.SemaphoreType.DMA,
        ],
    )
    return f(b8, src, dst, attr, w)


def _dec_body(p_ref, a2_ref, z_ref, wu_ref, bu_ref, wd_ref, bd_ref,
              wt_ref, bt_ref, h_ref, y_ref, t_ref):
    S = jnp.maximum(p_ref[0], p_ref[1])
    aggr = jnp.maximum(a2_ref[...] + S, 0.0)
    z = z_ref[...]
    h = jnp.dot(z, wu_ref[:H, :], preferred_element_type=jnp.float32)
    h = h + jnp.dot(aggr, wu_ref[H:, :], preferred_element_type=jnp.float32)
    h = jnp.maximum(h + bu_ref[...], 0.0)
    h_ref[...] = h
    y = jnp.dot(z, wd_ref[:H, :], preferred_element_type=jnp.float32)
    y = y + jnp.dot(h, wd_ref[H:, :], preferred_element_type=jnp.float32)
    y_ref[...] = jax.nn.sigmoid(y + bd_ref[...])
    hm = jnp.mean(h, axis=0, keepdims=True)
    wt = wt_ref[:H, :] + wt_ref[H:, :]
    t_ref[...] = jnp.dot(hm, wt, preferred_element_type=jnp.float32) + bt_ref[...]


def _decode(P, A2, z, W_U, b_U, W_dec, b_dec, W_ter, b_ter):
    return pl.pallas_call(
        _dec_body,
        out_shape=[
            jax.ShapeDtypeStruct((N, H), jnp.float32),
            jax.ShapeDtypeStruct((N, 1), jnp.float32),
            jax.ShapeDtypeStruct((1, 1), jnp.float32),
        ],
    )(P, A2, z, W_U, b_U.reshape(1, H), W_dec, b_dec.reshape(1, 1),
      W_ter, b_ter.reshape(1, 1))


def kernel(x, pre_h, edge_index, edge_attr, W_enc, b_enc, W_M, b_M,
           W_U, b_U, W_dec, b_dec, W_ter, b_ter):
    z, A2, B = _encode(x, pre_h, W_enc, b_enc, W_M, b_M)
    b8 = B.reshape(N * NHC, L)
    w = W_M[2 * H].reshape(NHC, L)
    src = edge_index[0]
    dst = edge_index[1]
    attr = edge_attr[:, 0]
    P = _segmax(b8, src, dst, attr, w)
    h, y, ter = _decode(P, A2, z, W_U, b_U, W_dec, b_dec, W_ter, b_ter)
    return (h, y, ter.reshape(()))
